# Initial kernel scaffold; baseline (speedup 1.0000x reference)
import jax
import jax.numpy as jnp
from jax.experimental import pallas as pl


def kernel(query, ref):
    raise NotImplementedError("write your pallas kernel here")


# speedup vs baseline: 13.7447x; 13.7447x over previous
import functools

import jax
import jax.numpy as jnp
from jax.experimental import pallas as pl


def _chamfer_body(q_ref, r_ref, out_ref, *, n_q_tiles, q_total):
    i = pl.program_id(0)

    q = q_ref[:, :]
    r = r_ref[:, :]

    q2 = jnp.sum(q * q, axis=1, keepdims=True)
    r2 = jnp.sum(r * r, axis=1)[None, :]

    dot = jax.lax.dot_general(
        q.astype(jnp.bfloat16),
        r.astype(jnp.bfloat16),
        dimension_numbers=(((1,), (1,)), ((), ())),
        preferred_element_type=jnp.float32,
    )

    d2 = q2 + r2 - 2.0 * dot
    row_min = jnp.min(d2, axis=1)
    tile_sum = jnp.sum(row_min).reshape(1, 1)

    @pl.when(i == 0)
    def _init():
        out_ref[:, :] = tile_sum

    @pl.when(i > 0)
    def _acc():
        out_ref[:, :] = out_ref[:, :] + tile_sum

    @pl.when(i == n_q_tiles - 1)
    def _finish():
        out_ref[:, :] = out_ref[:, :] / q_total


def kernel(query, ref):
    q_total, d = query.shape
    r_total, _ = ref.shape

    tile_q = 256 if q_total % 256 == 0 else q_total
    n_q_tiles = q_total // tile_q

    body = functools.partial(_chamfer_body, n_q_tiles=n_q_tiles,
                             q_total=float(q_total))
    out = pl.pallas_call(
        body,
        grid=(n_q_tiles,),
        in_specs=[
            pl.BlockSpec((tile_q, d), lambda i: (i, 0)),
            pl.BlockSpec((r_total, d), lambda i: (0, 0)),
        ],
        out_specs=pl.BlockSpec((1, 1), lambda i: (0, 0)),
        out_shape=jax.ShapeDtypeStruct((1, 1), jnp.float32),
    )(query, ref)
    return out[0, 0]


# speedup vs baseline: 20.4241x; 1.4860x over previous
import functools

import jax
import jax.numpy as jnp
from jax.experimental import pallas as pl
from jax.experimental.pallas import tpu as pltpu


def _chamfer_body(q_ref, r_ref, out_ref, rb_scratch, r2_scratch, *,
                  n_q_tiles, q_total):
    i = pl.program_id(0)

    @pl.when(i == 0)
    def _prep():
        r = r_ref[:, :]
        rb_scratch[:, :] = r.astype(jnp.bfloat16)
        r2_scratch[:, :] = jnp.sum(r * r, axis=1)[None, :]

    q = q_ref[:, :]
    q2 = jnp.sum(q * q, axis=1)

    dot = jax.lax.dot_general(
        q.astype(jnp.bfloat16),
        rb_scratch[:, :],
        dimension_numbers=(((1,), (1,)), ((), ())),
        preferred_element_type=jnp.float32,
    )

    t = r2_scratch[:, :] - 2.0 * dot
    row_min = jnp.min(t, axis=1) + q2
    tile_sum = jnp.sum(row_min).reshape(1, 1)

    @pl.when(i == 0)
    def _init():
        out_ref[:, :] = tile_sum

    @pl.when(i > 0)
    def _acc():
        out_ref[:, :] = out_ref[:, :] + tile_sum

    @pl.when(i == n_q_tiles - 1)
    def _finish():
        out_ref[:, :] = out_ref[:, :] / q_total


def kernel(query, ref):
    q_total, d = query.shape
    r_total, _ = ref.shape

    tile_q = 256 if q_total % 256 == 0 else q_total
    n_q_tiles = q_total // tile_q

    body = functools.partial(_chamfer_body, n_q_tiles=n_q_tiles,
                             q_total=float(q_total))
    out = pl.pallas_call(
        body,
        grid=(n_q_tiles,),
        in_specs=[
            pl.BlockSpec((tile_q, d), lambda i: (i, 0)),
            pl.BlockSpec((r_total, d), lambda i: (0, 0)),
        ],
        out_specs=pl.BlockSpec((1, 1), lambda i: (0, 0)),
        out_shape=jax.ShapeDtypeStruct((1, 1), jnp.float32),
        scratch_shapes=[
            pltpu.VMEM((r_total, d), jnp.bfloat16),
            pltpu.VMEM((1, r_total), jnp.float32),
        ],
    )(query, ref)
    return out[0, 0]


# speedup vs baseline: 22.6762x; 1.1103x over previous
import functools

import jax
import jax.numpy as jnp
from jax.experimental import pallas as pl
from jax.experimental.pallas import tpu as pltpu


def _chamfer_body(q_ref, r_ref, out_ref, raug_scratch, *, n_q_tiles, q_total):
    i = pl.program_id(0)

    @pl.when(i == 0)
    def _prep():
        r = r_ref[:, :]
        raug_scratch[:, : r.shape[1]] = r.astype(jnp.bfloat16)
        raug_scratch[:, r.shape[1]:] = (r * r).astype(jnp.bfloat16)

    q = q_ref[:, :]
    q2 = jnp.sum(q * q, axis=1)
    q_aug = jnp.concatenate(
        [q * -2.0, jnp.ones_like(q)], axis=1).astype(jnp.bfloat16)

    t = jax.lax.dot_general(
        q_aug,
        raug_scratch[:, :],
        dimension_numbers=(((1,), (1,)), ((), ())),
        preferred_element_type=jnp.float32,
    )

    row_min = jnp.min(t, axis=1) + q2
    tile_sum = jnp.sum(row_min).reshape(1, 1)

    @pl.when(i == 0)
    def _init():
        out_ref[:, :] = tile_sum

    @pl.when(i > 0)
    def _acc():
        out_ref[:, :] = out_ref[:, :] + tile_sum

    @pl.when(i == n_q_tiles - 1)
    def _finish():
        out_ref[:, :] = out_ref[:, :] / q_total


def kernel(query, ref):
    q_total, d = query.shape
    r_total, _ = ref.shape

    tile_q = 256 if q_total % 256 == 0 else q_total
    n_q_tiles = q_total // tile_q

    body = functools.partial(_chamfer_body, n_q_tiles=n_q_tiles,
                             q_total=float(q_total))
    out = pl.pallas_call(
        body,
        grid=(n_q_tiles,),
        in_specs=[
            pl.BlockSpec((tile_q, d), lambda i: (i, 0)),
            pl.BlockSpec((r_total, d), lambda i: (0, 0)),
        ],
        out_specs=pl.BlockSpec((1, 1), lambda i: (0, 0)),
        out_shape=jax.ShapeDtypeStruct((1, 1), jnp.float32),
        scratch_shapes=[
            pltpu.VMEM((r_total, 2 * d), jnp.bfloat16),
        ],
    )(query, ref)
    return out[0, 0]
